# baseline (speedup 1.0000x reference)
import functools

import jax
import jax.numpy as jnp
from jax import lax
from jax.experimental import pallas as pl
from jax.experimental.pallas import tpu as pltpu
from jax.experimental.pallas import tpu_sc as plsc

_N_LAYERS = 24
_N_HEADS = 32
_HEAD_DIM = 64
_PRE_SEQ_LEN = 64
_BATCH = 4
_ROW_STRIDE = _N_LAYERS * 2 * _N_HEADS
_ROWS_PER_OUT = 2 * _BATCH * _N_HEADS * _PRE_SEQ_LEN
_NW = 32
_ROWS_PER_W = _ROWS_PER_OUT // _NW
_CHUNK = 128
_NCHUNK = _ROWS_PER_W // _CHUNK


def _body(prefix_hbm, table_hbm, *refs):
    outs = refs[:_N_LAYERS]
    pref_v, idx_v, buf_a, buf_b, gsem, ssem = refs[_N_LAYERS:]
    bufs = (buf_a, buf_b)

    wid = lax.axis_index("s") * 2 + lax.axis_index("c")
    kv = wid // 16
    b = (wid // 4) % 4
    h0 = (wid % 4) * 8
    woff = kv * 32 + h0 - 64

    pltpu.sync_copy(prefix_hbm.at[b], pref_v)

    for g in range(8):
        for j in range(4):
            sl = pl.ds(j * 16, 16)
            idx_v[g, sl] = pref_v[sl] * _ROW_STRIDE + (woff + g)

    for i in range(_N_LAYERS):
        out_i = outs[i]
        lb = bufs[i % 2]
        out_slice = out_i.at[kv, b, pl.ds(h0, 8)]

        if i >= 2:
            prev_out = outs[i - 2]
            pltpu.make_async_copy(
                lb, prev_out.at[kv, b, pl.ds(h0, 8)],
                ssem.at[i % 2]).wait()

        @pl.loop(0, 8)
        def _fire(g):
            for j in range(4):
                sl = pl.ds(j * 16, 16)
                idx_v[g, sl] = idx_v[g, sl] + 64
            pltpu.async_copy(table_hbm.at[idx_v.at[g]], lb.at[g], gsem)

        pltpu.make_async_copy(out_slice, lb, gsem).wait()

        pltpu.async_copy(lb, out_slice, ssem.at[i % 2])

    for i in (_N_LAYERS - 2, _N_LAYERS - 1):
        pltpu.make_async_copy(
            bufs[i % 2], outs[i].at[kv, b, pl.ds(h0, 8)],
            ssem.at[i % 2]).wait()


@functools.partial(jax.jit, static_argnames=())
def _sc_gather(prefix, table_r):
    mesh = plsc.VectorSubcoreMesh(core_axis_name="c", subcore_axis_name="s")
    out_type = [jax.ShapeDtypeStruct(
        (2, _BATCH, _N_HEADS, _PRE_SEQ_LEN, _HEAD_DIM), jnp.float32)
                for _ in range(_N_LAYERS)]
    scratch = [
        pltpu.VMEM((_PRE_SEQ_LEN,), jnp.int32),
        pltpu.VMEM((8, _PRE_SEQ_LEN), jnp.int32),
        pltpu.VMEM((8, _PRE_SEQ_LEN, _HEAD_DIM), jnp.float32),
        pltpu.VMEM((8, _PRE_SEQ_LEN, _HEAD_DIM), jnp.float32),
        pltpu.SemaphoreType.DMA,
        pltpu.SemaphoreType.DMA((2,)),
    ]
    return pl.kernel(
        _body, out_type=out_type, mesh=mesh, scratch_types=scratch,
        compiler_params=pltpu.CompilerParams(use_tc_tiling_on_sc=False),
    )(prefix, table_r)


def kernel(prefix, table):
    table_r = table.reshape(_PRE_SEQ_LEN * _ROW_STRIDE, _HEAD_DIM)
    return tuple(_sc_gather(prefix, table_r))


# speedup vs baseline: 1.0705x; 1.0705x over previous
import functools

import jax
import jax.numpy as jnp
from jax import lax
from jax.experimental import pallas as pl
from jax.experimental.pallas import tpu as pltpu
from jax.experimental.pallas import tpu_sc as plsc

_N_LAYERS = 24
_N_HEADS = 32
_HEAD_DIM = 64
_PRE_SEQ_LEN = 64
_BATCH = 4
_ROW_STRIDE = _N_LAYERS * 2 * _N_HEADS
_N_CALLS = 4
_LAYERS_PER_CALL = _N_LAYERS // _N_CALLS


def _make_body(lo):
    n_layers = _LAYERS_PER_CALL

    def _body(prefix_hbm, table_hbm, *refs):
        outs = refs[:n_layers]
        pref_v, idx_v, buf_a, buf_b, gsem, ssem = refs[n_layers:]
        bufs = (buf_a, buf_b)

        wid = lax.axis_index("s") * 2 + lax.axis_index("c")
        kv = wid // 16
        b = (wid // 4) % 4
        h0 = (wid % 4) * 8
        woff = kv * 32 + h0 + (lo - 1) * 64

        pltpu.sync_copy(prefix_hbm.at[b], pref_v)

        for g in range(8):
            for j in range(4):
                sl = pl.ds(j * 16, 16)
                idx_v[g, sl] = pref_v[sl] * _ROW_STRIDE + (woff + g)

        for i in range(n_layers):
            out_i = outs[i]
            lb = bufs[i % 2]
            out_slice = out_i.at[kv, b, pl.ds(h0, 8)]

            if i >= 2:
                pltpu.make_async_copy(
                    lb, outs[i - 2].at[kv, b, pl.ds(h0, 8)],
                    ssem.at[i % 2]).wait()

            @pl.loop(0, 8)
            def _fire(g):
                for j in range(4):
                    sl = pl.ds(j * 16, 16)
                    idx_v[g, sl] = idx_v[g, sl] + 64
                pltpu.async_copy(table_hbm.at[idx_v.at[g]], lb.at[g], gsem)

            pltpu.make_async_copy(out_slice, lb, gsem).wait()

            pltpu.async_copy(lb, out_slice, ssem.at[i % 2])

        for i in (n_layers - 2, n_layers - 1):
            pltpu.make_async_copy(
                bufs[i % 2], outs[i].at[kv, b, pl.ds(h0, 8)],
                ssem.at[i % 2]).wait()

    return _body


@functools.partial(jax.jit, static_argnames=())
def _sc_gather(prefix, table_r):
    mesh = plsc.VectorSubcoreMesh(core_axis_name="c", subcore_axis_name="s")
    out_type = [jax.ShapeDtypeStruct(
        (2, _BATCH, _N_HEADS, _PRE_SEQ_LEN, _HEAD_DIM), jnp.float32)
                for _ in range(_LAYERS_PER_CALL)]
    scratch = [
        pltpu.VMEM((_PRE_SEQ_LEN,), jnp.int32),
        pltpu.VMEM((8, _PRE_SEQ_LEN), jnp.int32),
        pltpu.VMEM((8, _PRE_SEQ_LEN, _HEAD_DIM), jnp.float32),
        pltpu.VMEM((8, _PRE_SEQ_LEN, _HEAD_DIM), jnp.float32),
        pltpu.SemaphoreType.DMA,
        pltpu.SemaphoreType.DMA((2,)),
    ]
    outs = []
    for call in range(_N_CALLS):
        outs.extend(pl.kernel(
            _make_body(call * _LAYERS_PER_CALL),
            out_type=out_type, mesh=mesh, scratch_types=scratch,
            compiler_params=pltpu.CompilerParams(use_tc_tiling_on_sc=False),
        )(prefix, table_r))
    return outs


def kernel(prefix, table):
    table_r = table.reshape(_PRE_SEQ_LEN * _ROW_STRIDE, _HEAD_DIM)
    return tuple(_sc_gather(prefix, table_r))
